# baseline (speedup 1.0000x reference)
import functools
import math

import jax
import jax.numpy as jnp
from jax import lax
from jax.experimental import pallas as pl
from jax.experimental.pallas import tpu as pltpu
from jax.experimental.pallas import tpu_sc as plsc

_C = 1000
_PAD = 0
_SMOOTH = 0.1
_CONF = 1.0 - _SMOOTH
_EPS = _SMOOTH / (_C - 2)
_K = (_C - 2) * _EPS * math.log(_EPS) + _CONF * math.log(_CONF)

_L = 16
_NC, _NS = 2, 16
_NW = _NC * _NS
_SLAB = 16
_NBUF = 4
_N_FULL = (_C // _L) * _L - _L
_TAIL = _C - _L


def _make_sc_loss(b_dim, t_dim, t_split):
    w_per_b = _NW // b_dim
    rows_per_w = (t_dim - t_split) // w_per_b
    n_slabs = rows_per_w // _SLAB

    @functools.partial(
        pl.kernel,
        mesh=plsc.VectorSubcoreMesh(core_axis_name="c", subcore_axis_name="s"),
        out_type=jax.ShapeDtypeStruct((_NW, _L), jnp.float32),
        scratch_types=[
            pltpu.VMEM((rows_per_w * _L,), jnp.int32),
            pltpu.VMEM((_NBUF, _SLAB, _C), jnp.float32),
            pltpu.VMEM((_L,), jnp.float32),
        ] + [pltpu.SemaphoreType.DMA] * _NBUF,
    )
    def sc_loss(pred_hbm, goldrep_hbm, out_hbm, gold_v, buf_v, acc_v,
                *sems):
        wid = lax.axis_index("s") * _NC + lax.axis_index("c")
        bidx = wid // w_per_b
        trow0 = t_split + (wid % w_per_b) * rows_per_w
        row0 = bidx * t_dim + trow0

        pltpu.sync_copy(goldrep_hbm.at[pl.ds(row0 * _L, rows_per_w * _L)],
                        gold_v)

        def start(s):
            sl = s % _NBUF
            return pltpu.async_copy(
                pred_hbm.at[bidx, pl.ds(trow0 + s * _SLAB, _SLAB), :],
                buf_v.at[sl], sems[sl])

        lane = lax.iota(jnp.int32, _L)
        zero16 = jnp.zeros((_L,), jnp.float32)
        lane0f = jnp.where(lane == 0, 1.0, 0.0)
        tail_keep = lane >= ((_N_FULL + _L) - _TAIL)
        lane_hi = jnp.where(lane >= ((_N_FULL + _L) - _TAIL), lane,
                            jnp.full((_L,), 2 * _L, jnp.int32))
        konst = lane0f * jnp.float32(_K)
        epslane = lane0f * jnp.float32(_EPS)

        def process(s, carry):
            sl = s % _NBUF
            slab = buf_v.at[sl]

            def row_body(r, carry):
                S, G = carry
                off = pl.multiple_of((s * _SLAB + r) * _L, _L)
                gr_vec = gold_v[pl.ds(off, _L)]
                gr = gr_vec[0]
                mrow = gr_vec != _PAD

                first = slab[r, pl.ds(0, _L)]
                accs = [first, zero16, zero16, zero16]
                for i, c in enumerate(range(_L, _N_FULL + 1, _L)):
                    accs[(i + 1) % 4] = accs[(i + 1) % 4] + slab[r, pl.ds(c, _L)]
                tail = slab[r, pl.ds(_TAIL, _L)]
                acc = ((accs[0] + accs[1]) + (accs[2] + accs[3])
                       + jnp.where(tail_keep, tail, zero16))

                start_c = pl.multiple_of(
                    (jnp.minimum(gr, _N_FULL + _L - 1) // _L) * _L, _L)
                window = slab[r, pl.ds(start_c, _L)]
                grs = jnp.full((_L,), gr, jnp.int32)
                pg_w = jnp.where(lane == grs - start_c, window, zero16)
                pg_t = jnp.where(lane_hi == grs - _TAIL, tail, zero16)

                S = S + jnp.where(mrow, acc, zero16)
                G = G + jnp.where(
                    mrow,
                    konst + epslane * first
                    + jnp.float32(_EPS - _CONF) * (pg_w + pg_t),
                    zero16)
                return S, G

            @plsc.parallel_loop(0, _SLAB, carry=carry, unroll=4)
            def new_carry(r, c):
                return row_body(r, c)
            return new_carry

        cps = [None] * _NBUF
        for s in range(_NBUF - 1):
            cps[s] = start(s)
        carry = (zero16, zero16)
        for s in range(n_slabs):
            if s + _NBUF - 1 < n_slabs:
                cps[(s + _NBUF - 1) % _NBUF] = start(s + _NBUF - 1)
            cps[s % _NBUF].wait()
            carry = process(s, carry)

        S, G = carry
        acc_v[...] = G - jnp.float32(_EPS) * S
        pltpu.sync_copy(acc_v, out_hbm.at[wid])

    return sc_loss


_T_SPLIT = 512


def _tc_body(g_ref, p_ref, out_ref):
    i = pl.program_id(0)

    p = p_ref[0]
    g = g_ref[0]
    mask = g != _PAD

    rowsum = jnp.sum(p, axis=1, keepdims=True)
    p0 = p[:, 0:1]
    col = jax.lax.broadcasted_iota(jnp.int32, p.shape, 1)
    pg = jnp.sum(jnp.where(col == g, p, 0.0), axis=1, keepdims=True)

    per_row = _K - (_EPS * (rowsum - p0 - pg) + _CONF * pg)
    blk = jnp.sum(jnp.where(mask, per_row, 0.0), keepdims=True).reshape(1, 1)

    @pl.when(i == 0)
    def _():
        out_ref[...] = jnp.zeros_like(out_ref)

    out_ref[...] += blk


def _tc_loss(pred, gold3, t_split):
    b, t, c = pred.shape
    out = pl.pallas_call(
        _tc_body,
        grid=(b,),
        in_specs=[
            pl.BlockSpec((1, t_split, 1), lambda i: (i, 0, 0)),
            pl.BlockSpec((1, t_split, c), lambda i: (i, 0, 0)),
        ],
        out_specs=pl.BlockSpec((1, 1), lambda i: (0, 0)),
        out_shape=jax.ShapeDtypeStruct((1, 1), jnp.float32),
    )(gold3, pred)
    return out[0, 0]


def kernel(pred, gold):
    b, t, c = pred.shape
    g = gold.reshape(-1).astype(jnp.int32)
    g_rep = jnp.broadcast_to(g[:, None], (b * t, _L)).reshape(-1)
    sc_part = _make_sc_loss(b, t, _T_SPLIT)(pred, g_rep)
    tc_part = _tc_loss(pred, g.reshape(b, t, 1), _T_SPLIT)
    return tc_part + jnp.sum(sc_part)


# speedup vs baseline: 1.0475x; 1.0475x over previous
import functools
import math

import jax
import jax.numpy as jnp
from jax import lax
from jax.experimental import pallas as pl
from jax.experimental.pallas import tpu as pltpu
from jax.experimental.pallas import tpu_sc as plsc

_C = 1000
_PAD = 0
_SMOOTH = 0.1
_CONF = 1.0 - _SMOOTH
_EPS = _SMOOTH / (_C - 2)
_K = (_C - 2) * _EPS * math.log(_EPS) + _CONF * math.log(_CONF)

_L = 16
_NC, _NS = 2, 16
_NW = _NC * _NS
_SLAB = 16
_NBUF = 4
_N_FULL = (_C // _L) * _L - _L
_TAIL = _C - _L


def _make_sc_loss(b_dim, t_dim, t_split):
    w_per_b = _NW // b_dim
    rows_per_w = (t_dim - t_split) // w_per_b
    n_slabs = rows_per_w // _SLAB

    @functools.partial(
        pl.kernel,
        mesh=plsc.VectorSubcoreMesh(core_axis_name="c", subcore_axis_name="s"),
        out_type=jax.ShapeDtypeStruct((_NW, _L), jnp.float32),
        scratch_types=[
            pltpu.VMEM((rows_per_w * _L,), jnp.int32),
            pltpu.VMEM((_NBUF, _SLAB, _C), jnp.float32),
            pltpu.VMEM((_L,), jnp.float32),
        ] + [pltpu.SemaphoreType.DMA] * _NBUF,
    )
    def sc_loss(pred_hbm, goldrep_hbm, out_hbm, gold_v, buf_v, acc_v,
                *sems):
        wid = lax.axis_index("s") * _NC + lax.axis_index("c")
        bidx = wid // w_per_b
        trow0 = t_split + (wid % w_per_b) * rows_per_w
        row0 = bidx * t_dim + trow0

        pltpu.sync_copy(goldrep_hbm.at[pl.ds(row0 * _L, rows_per_w * _L)],
                        gold_v)

        def start(s):
            sl = s % _NBUF
            return pltpu.async_copy(
                pred_hbm.at[bidx, pl.ds(trow0 + s * _SLAB, _SLAB), :],
                buf_v.at[sl], sems[sl])

        lane = lax.iota(jnp.int32, _L)
        zero16 = jnp.zeros((_L,), jnp.float32)
        lane0f = jnp.where(lane == 0, 1.0, 0.0)
        tail_keep = lane >= ((_N_FULL + _L) - _TAIL)
        lane_hi = jnp.where(lane >= ((_N_FULL + _L) - _TAIL), lane,
                            jnp.full((_L,), 2 * _L, jnp.int32))
        konst = lane0f * jnp.float32(_K)
        epslane = lane0f * jnp.float32(_EPS)

        def process(s, carry):
            sl = s % _NBUF
            slab = buf_v.at[sl]

            def row_body(r, carry):
                S, G = carry
                off = pl.multiple_of((s * _SLAB + r) * _L, _L)
                gr_vec = gold_v[pl.ds(off, _L)]
                gr = gr_vec[0]
                mrow = gr_vec != _PAD

                first = slab[r, pl.ds(0, _L)]
                accs = [first, zero16, zero16, zero16]
                for i, c in enumerate(range(_L, _N_FULL + 1, _L)):
                    accs[(i + 1) % 4] = accs[(i + 1) % 4] + slab[r, pl.ds(c, _L)]
                tail = slab[r, pl.ds(_TAIL, _L)]
                acc = ((accs[0] + accs[1]) + (accs[2] + accs[3])
                       + jnp.where(tail_keep, tail, zero16))

                start_c = pl.multiple_of(
                    (jnp.minimum(gr, _N_FULL + _L - 1) // _L) * _L, _L)
                window = slab[r, pl.ds(start_c, _L)]
                grs = jnp.full((_L,), gr, jnp.int32)
                pg_w = jnp.where(lane == grs - start_c, window, zero16)
                pg_t = jnp.where(lane_hi == grs - _TAIL, tail, zero16)

                S = S + jnp.where(mrow, acc, zero16)
                G = G + jnp.where(
                    mrow,
                    konst + epslane * first
                    + jnp.float32(_EPS - _CONF) * (pg_w + pg_t),
                    zero16)
                return S, G

            @plsc.parallel_loop(0, _SLAB, carry=carry, unroll=4)
            def new_carry(r, c):
                return row_body(r, c)
            return new_carry

        cps = [None] * _NBUF
        for s in range(_NBUF - 1):
            cps[s] = start(s)
        carry = (zero16, zero16)
        for s in range(n_slabs):
            if s + _NBUF - 1 < n_slabs:
                cps[(s + _NBUF - 1) % _NBUF] = start(s + _NBUF - 1)
            cps[s % _NBUF].wait()
            carry = process(s, carry)

        S, G = carry
        acc_v[...] = G - jnp.float32(_EPS) * S
        pltpu.sync_copy(acc_v, out_hbm.at[wid])

    return sc_loss


_T_SPLIT = 1536


def _tc_body(g_ref, p_ref, out_ref):
    i = pl.program_id(0)

    p = p_ref[0]
    g = g_ref[0]
    mask = g != _PAD

    rowsum = jnp.sum(p, axis=1, keepdims=True)
    p0 = p[:, 0:1]
    col = jax.lax.broadcasted_iota(jnp.int32, p.shape, 1)
    pg = jnp.sum(jnp.where(col == g, p, 0.0), axis=1, keepdims=True)

    per_row = _K - (_EPS * (rowsum - p0 - pg) + _CONF * pg)
    blk = jnp.sum(jnp.where(mask, per_row, 0.0), keepdims=True).reshape(1, 1)

    @pl.when(i == 0)
    def _():
        out_ref[...] = jnp.zeros_like(out_ref)

    out_ref[...] += blk


def _tc_loss(pred, gold3, t_split):
    b, t, c = pred.shape
    out = pl.pallas_call(
        _tc_body,
        grid=(b,),
        in_specs=[
            pl.BlockSpec((1, t_split, 1), lambda i: (i, 0, 0)),
            pl.BlockSpec((1, t_split, c), lambda i: (i, 0, 0)),
        ],
        out_specs=pl.BlockSpec((1, 1), lambda i: (0, 0)),
        out_shape=jax.ShapeDtypeStruct((1, 1), jnp.float32),
    )(gold3, pred)
    return out[0, 0]


def kernel(pred, gold):
    b, t, c = pred.shape
    g = gold.reshape(-1).astype(jnp.int32)
    g_rep = jnp.broadcast_to(g[:, None], (b * t, _L)).reshape(-1)
    sc_part = _make_sc_loss(b, t, _T_SPLIT)(pred, g_rep)
    tc_part = _tc_loss(pred, g.reshape(b, t, 1), _T_SPLIT)
    return tc_part + jnp.sum(sc_part)


# speedup vs baseline: 1.0576x; 1.0096x over previous
import functools
import math

import jax
import jax.numpy as jnp
from jax import lax
from jax.experimental import pallas as pl
from jax.experimental.pallas import tpu as pltpu
from jax.experimental.pallas import tpu_sc as plsc

_C = 1000
_PAD = 0
_SMOOTH = 0.1
_CONF = 1.0 - _SMOOTH
_EPS = _SMOOTH / (_C - 2)
_K = (_C - 2) * _EPS * math.log(_EPS) + _CONF * math.log(_CONF)

_L = 16
_NC, _NS = 2, 16
_NW = _NC * _NS
_SLAB = 16
_NBUF = 4
_N_FULL = (_C // _L) * _L - _L
_TAIL = _C - _L


def _make_sc_loss(b_dim, t_dim, t_split):
    w_per_b = _NW // b_dim
    rows_per_w = (t_dim - t_split) // w_per_b
    n_slabs = rows_per_w // _SLAB

    @functools.partial(
        pl.kernel,
        mesh=plsc.VectorSubcoreMesh(core_axis_name="c", subcore_axis_name="s"),
        out_type=jax.ShapeDtypeStruct((_NW, _L), jnp.float32),
        scratch_types=[
            pltpu.VMEM((rows_per_w * _L,), jnp.int32),
            pltpu.VMEM((_NBUF, _SLAB, _C), jnp.float32),
            pltpu.VMEM((_L,), jnp.float32),
        ] + [pltpu.SemaphoreType.DMA] * _NBUF,
    )
    def sc_loss(pred_hbm, goldrep_hbm, out_hbm, gold_v, buf_v, acc_v,
                *sems):
        wid = lax.axis_index("s") * _NC + lax.axis_index("c")
        bidx = wid // w_per_b
        trow0 = t_split + (wid % w_per_b) * rows_per_w
        row0 = bidx * t_dim + trow0

        pltpu.sync_copy(goldrep_hbm.at[pl.ds(row0 * _L, rows_per_w * _L)],
                        gold_v)

        def start(s):
            sl = s % _NBUF
            return pltpu.async_copy(
                pred_hbm.at[bidx, pl.ds(trow0 + s * _SLAB, _SLAB), :],
                buf_v.at[sl], sems[sl])

        lane = lax.iota(jnp.int32, _L)
        zero16 = jnp.zeros((_L,), jnp.float32)
        lane0f = jnp.where(lane == 0, 1.0, 0.0)
        tail_keep = lane >= ((_N_FULL + _L) - _TAIL)
        lane_hi = jnp.where(lane >= ((_N_FULL + _L) - _TAIL), lane,
                            jnp.full((_L,), 2 * _L, jnp.int32))
        konst = lane0f * jnp.float32(_K)
        epslane = lane0f * jnp.float32(_EPS)

        def process(s, carry):
            sl = s % _NBUF
            slab = buf_v.at[sl]

            def row_body(r, carry):
                S, G = carry
                off = pl.multiple_of((s * _SLAB + r) * _L, _L)
                gr_vec = gold_v[pl.ds(off, _L)]
                gr = gr_vec[0]
                mrow = gr_vec != _PAD

                first = slab[r, pl.ds(0, _L)]
                accs = [first, zero16, zero16, zero16]
                for i, c in enumerate(range(_L, _N_FULL + 1, _L)):
                    accs[(i + 1) % 4] = accs[(i + 1) % 4] + slab[r, pl.ds(c, _L)]
                tail = slab[r, pl.ds(_TAIL, _L)]
                acc = ((accs[0] + accs[1]) + (accs[2] + accs[3])
                       + jnp.where(tail_keep, tail, zero16))

                start_c = pl.multiple_of(
                    (jnp.minimum(gr, _N_FULL + _L - 1) // _L) * _L, _L)
                window = slab[r, pl.ds(start_c, _L)]
                grs = jnp.full((_L,), gr, jnp.int32)
                pg_w = jnp.where(lane == grs - start_c, window, zero16)
                pg_t = jnp.where(lane_hi == grs - _TAIL, tail, zero16)

                S = S + jnp.where(mrow, acc, zero16)
                G = G + jnp.where(
                    mrow,
                    konst + epslane * first
                    + jnp.float32(_EPS - _CONF) * (pg_w + pg_t),
                    zero16)
                return S, G

            @plsc.parallel_loop(0, _SLAB, carry=carry, unroll=4)
            def new_carry(r, c):
                return row_body(r, c)
            return new_carry

        cps = [None] * _NBUF
        for s in range(_NBUF - 1):
            cps[s] = start(s)
        carry = (zero16, zero16)
        for s in range(n_slabs):
            if s + _NBUF - 1 < n_slabs:
                cps[(s + _NBUF - 1) % _NBUF] = start(s + _NBUF - 1)
            cps[s % _NBUF].wait()
            carry = process(s, carry)

        S, G = carry
        acc_v[...] = G - jnp.float32(_EPS) * S
        pltpu.sync_copy(acc_v, out_hbm.at[wid])

    return sc_loss


_T_SPLIT = 1152


def _tc_body(g_ref, p_ref, out_ref):
    i = pl.program_id(0)

    p = p_ref[0]
    g = g_ref[0]
    mask = g != _PAD

    rowsum = jnp.sum(p, axis=1, keepdims=True)
    p0 = p[:, 0:1]
    col = jax.lax.broadcasted_iota(jnp.int32, p.shape, 1)
    pg = jnp.sum(jnp.where(col == g, p, 0.0), axis=1, keepdims=True)

    per_row = _K - (_EPS * (rowsum - p0 - pg) + _CONF * pg)
    blk = jnp.sum(jnp.where(mask, per_row, 0.0), keepdims=True).reshape(1, 1)

    @pl.when(i == 0)
    def _():
        out_ref[...] = jnp.zeros_like(out_ref)

    out_ref[...] += blk


def _tc_loss(pred, gold3, t_split):
    b, t, c = pred.shape
    out = pl.pallas_call(
        _tc_body,
        grid=(b,),
        in_specs=[
            pl.BlockSpec((1, t_split, 1), lambda i: (i, 0, 0)),
            pl.BlockSpec((1, t_split, c), lambda i: (i, 0, 0)),
        ],
        out_specs=pl.BlockSpec((1, 1), lambda i: (0, 0)),
        out_shape=jax.ShapeDtypeStruct((1, 1), jnp.float32),
    )(gold3, pred)
    return out[0, 0]


def kernel(pred, gold):
    b, t, c = pred.shape
    g = gold.reshape(-1).astype(jnp.int32)
    g_rep = jnp.broadcast_to(g[:, None], (b * t, _L)).reshape(-1)
    sc_part = _make_sc_loss(b, t, _T_SPLIT)(pred, g_rep)
    tc_part = _tc_loss(pred, g.reshape(b, t, 1), _T_SPLIT)
    return tc_part + jnp.sum(sc_part)
